# baseline (speedup 1.0000x reference)
import jax
import jax.numpy as jnp
from jax import lax
from jax.experimental import pallas as pl
from jax.experimental.pallas import tpu as pltpu
from jax.experimental.pallas import tpu_sc as plsc

_VOCAB = 1000000
_EMBED = 50
_LANE = 128
_WINDOW = 5
_HIDDEN = 100
_TAGS = 46
_BATCH = 16384

_PB = 16384
_PBBITS = 14
_NPACK = 31
_TROWS = _NPACK * _PB

_NIDX = _BATCH * _WINDOW
_NC = 2
_NS = 16
_NW = _NC * _NS
_ROWS_W = _NIDX // _NW
_CHUNK = 320
_NCHUNKS = _ROWS_W // _CHUNK


def _pack_body(ea_ref, eb_ref, out_ref):
    i = pl.program_id(0)
    eye = jnp.eye(_EMBED, dtype=jnp.float32)
    dn = (((0,), (0,)), ((), ()))
    ta = lax.dot_general(ea_ref[...], eye, dn,
                         preferred_element_type=jnp.float32)
    tb = lax.dot_general(eb_ref[...], eye, dn,
                         preferred_element_type=jnp.float32)
    row2 = lax.broadcasted_iota(jnp.int32, (_PB, 1), 0)
    vb = ((2 * i + 1) * _PB + row2) < _VOCAB
    tb = jnp.where(vb, tb, 0.0)
    zeros14 = jnp.zeros((_PB, 64 - _EMBED), jnp.float32)
    out_ref[:, 0:_EMBED] = ta
    out_ref[:, _EMBED:64] = zeros14
    out_ref[:, 64:64 + _EMBED] = tb
    out_ref[:, 64 + _EMBED:_LANE] = zeros14


def _tc_pack(et):
    return pl.pallas_call(
        _pack_body,
        grid=(_NPACK,),
        in_specs=[
            pl.BlockSpec((_EMBED, _PB), lambda i: (0, 2 * i)),
            pl.BlockSpec(
                (_EMBED, _PB),
                lambda i: (0, jnp.minimum(2 * i + 1, _VOCAB // _PB))),
        ],
        out_specs=pl.BlockSpec((_PB, _LANE), lambda i: (i, 0)),
        out_shape=jax.ShapeDtypeStruct((_TROWS, _LANE), jnp.float32),
    )(et, et)


def _sc_gather_body(table, idxs, out, idx_v, rows0, rows1, sem0, sem1):
    c = lax.axis_index("c")
    s = lax.axis_index("s")
    wid = s * _NC + c
    base = wid * _ROWS_W
    pltpu.sync_copy(idxs.at[pl.ds(base, _ROWS_W)], idx_v)
    bufs = (rows0, rows1)
    sems = (sem0, sem1)
    copies = [None, None]
    copies[0] = pltpu.async_copy(
        table.at[idx_v.at[pl.ds(0, _CHUNK)]], bufs[0], sems[0])
    for j in range(_NCHUNKS):
        cur = j % 2
        nxt = (j + 1) % 2
        if j + 1 < _NCHUNKS:
            copies[nxt] = pltpu.async_copy(
                table.at[idx_v.at[pl.ds((j + 1) * _CHUNK, _CHUNK)]],
                bufs[nxt], sems[nxt])
        copies[cur].wait()
        pltpu.sync_copy(bufs[cur], out.at[pl.ds(base + j * _CHUNK, _CHUNK)])


_sc_gather = pl.kernel(
    _sc_gather_body,
    out_type=jax.ShapeDtypeStruct((_NIDX, _LANE), jnp.float32),
    scratch_types=[
        pltpu.VMEM((_ROWS_W,), jnp.int32),
        pltpu.VMEM((_CHUNK, _LANE), jnp.float32),
        pltpu.VMEM((_CHUNK, _LANE), jnp.float32),
        pltpu.SemaphoreType.DMA,
        pltpu.SemaphoreType.DMA,
    ],
    mesh=plsc.VectorSubcoreMesh(core_axis_name="c", subcore_axis_name="s"),
    compiler_params=pltpu.CompilerParams(use_tc_tiling_on_sc=True),
)


_BS = 2048


def _mlp_body(h_ref, m_ref, w0_ref, b0_ref, w1_ref, b1_ref,
              out_ref):
    h = None
    for w in range(_WINDOW):
        g = h_ref[w]
        m = m_ref[:, w:w + 1]
        w0w = w0_ref[:, pl.ds(_EMBED * w, _EMBED)]
        dn = (((1,), (1,)), ((), ()))
        ha = lax.dot_general(g[:, 0:_EMBED], w0w, dn,
                             preferred_element_type=jnp.float32)
        hb = lax.dot_general(g[:, 64:64 + _EMBED], w0w, dn,
                             preferred_element_type=jnp.float32)
        hw = ha + m * (hb - ha)
        h = hw if h is None else h + hw
    h = jnp.tanh(h + b0_ref[...])
    logits = lax.dot_general(h, w1_ref[...], (((1,), (1,)), ((), ())),
                             preferred_element_type=jnp.float32)
    logits = logits + b1_ref[...]
    mx = jnp.max(logits, axis=1, keepdims=True)
    sh = logits - mx
    out_ref[...] = sh - jnp.log(jnp.sum(jnp.exp(sh), axis=1, keepdims=True))


def _tc_mlp(h5, mpk, w0, b0, w1, b1):
    return pl.pallas_call(
        _mlp_body,
        grid=(_BATCH // _BS,),
        in_specs=[
            pl.BlockSpec((_WINDOW, _BS, _LANE), lambda i: (0, i, 0)),
            pl.BlockSpec((_BS, 8), lambda i: (i, 0)),
            pl.BlockSpec((_HIDDEN, _WINDOW * _EMBED), lambda i: (0, 0)),
            pl.BlockSpec((1, _HIDDEN), lambda i: (0, 0)),
            pl.BlockSpec((_TAGS, _HIDDEN), lambda i: (0, 0)),
            pl.BlockSpec((1, _TAGS), lambda i: (0, 0)),
        ],
        out_specs=pl.BlockSpec((_BS, _TAGS), lambda i: (i, 0)),
        out_shape=jax.ShapeDtypeStruct((_BATCH, _TAGS), jnp.float32),
    )(h5, mpk, w0, b0, w1, b1)


def kernel(x, E, W0, b0, W1, b1):
    idx = jnp.asarray(x, jnp.int32).T.reshape(-1)
    q = idx >> _PBBITS
    r = idx & (_PB - 1)
    tpk = ((q >> 1) << _PBBITS) + r
    half = q & 1
    mpk = jnp.pad(
        half.reshape(_WINDOW, _BATCH).T.astype(jnp.float32),
        ((0, 0), (0, 8 - _WINDOW)))

    Epk = _tc_pack(E.T)
    emb = _sc_gather(Epk, tpk)
    h5 = emb.reshape(_WINDOW, _BATCH, _LANE)
    return _tc_mlp(h5, mpk, W0.T,
                   b0.reshape(1, _HIDDEN), W1.T, b1.reshape(1, _TAGS))


# speedup vs baseline: 1.0002x; 1.0002x over previous
import jax
import jax.numpy as jnp
from jax import lax
from jax.experimental import pallas as pl
from jax.experimental.pallas import tpu as pltpu
from jax.experimental.pallas import tpu_sc as plsc

_VOCAB = 1000000
_EMBED = 50
_LANE = 128
_WINDOW = 5
_HIDDEN = 100
_TAGS = 46
_BATCH = 16384

_PB = 16384
_PBBITS = 14
_NPACK = 31
_TROWS = _NPACK * _PB

_NIDX = _BATCH * _WINDOW
_NC = 2
_NS = 16
_NW = _NC * _NS
_ROWS_W = _NIDX // _NW
_CHUNK = 320
_NCHUNKS = _ROWS_W // _CHUNK


def _pack_body(ea_ref, eb_ref, out_ref):
    i = pl.program_id(0)
    eye = jnp.eye(_EMBED, dtype=jnp.float32)
    dn = (((0,), (0,)), ((), ()))
    ta = lax.dot_general(ea_ref[...], eye, dn,
                         preferred_element_type=jnp.float32)
    tb = jnp.transpose(eb_ref[...], (1, 0))
    row2 = lax.broadcasted_iota(jnp.int32, (_PB, 1), 0)
    vb = ((2 * i + 1) * _PB + row2) < _VOCAB
    tb = jnp.where(vb, tb, 0.0)
    zeros14 = jnp.zeros((_PB, 64 - _EMBED), jnp.float32)
    out_ref[:, 0:_EMBED] = ta
    out_ref[:, _EMBED:64] = zeros14
    out_ref[:, 64:64 + _EMBED] = tb
    out_ref[:, 64 + _EMBED:_LANE] = zeros14


def _tc_pack(et):
    return pl.pallas_call(
        _pack_body,
        grid=(_NPACK,),
        in_specs=[
            pl.BlockSpec((_EMBED, _PB), lambda i: (0, 2 * i)),
            pl.BlockSpec(
                (_EMBED, _PB),
                lambda i: (0, jnp.minimum(2 * i + 1, _VOCAB // _PB))),
        ],
        out_specs=pl.BlockSpec((_PB, _LANE), lambda i: (i, 0)),
        out_shape=jax.ShapeDtypeStruct((_TROWS, _LANE), jnp.float32),
    )(et, et)


def _sc_gather_body(table, idxs, out, idx_v, rows0, rows1, sem0, sem1):
    c = lax.axis_index("c")
    s = lax.axis_index("s")
    wid = s * _NC + c
    base = wid * _ROWS_W
    pltpu.sync_copy(idxs.at[pl.ds(base, _ROWS_W)], idx_v)
    bufs = (rows0, rows1)
    sems = (sem0, sem1)
    copies = [None, None]
    copies[0] = pltpu.async_copy(
        table.at[idx_v.at[pl.ds(0, _CHUNK)]], bufs[0], sems[0])
    for j in range(_NCHUNKS):
        cur = j % 2
        nxt = (j + 1) % 2
        if j + 1 < _NCHUNKS:
            copies[nxt] = pltpu.async_copy(
                table.at[idx_v.at[pl.ds((j + 1) * _CHUNK, _CHUNK)]],
                bufs[nxt], sems[nxt])
        copies[cur].wait()
        pltpu.sync_copy(bufs[cur], out.at[pl.ds(base + j * _CHUNK, _CHUNK)])


_sc_gather = pl.kernel(
    _sc_gather_body,
    out_type=jax.ShapeDtypeStruct((_NIDX, _LANE), jnp.float32),
    scratch_types=[
        pltpu.VMEM((_ROWS_W,), jnp.int32),
        pltpu.VMEM((_CHUNK, _LANE), jnp.float32),
        pltpu.VMEM((_CHUNK, _LANE), jnp.float32),
        pltpu.SemaphoreType.DMA,
        pltpu.SemaphoreType.DMA,
    ],
    mesh=plsc.VectorSubcoreMesh(core_axis_name="c", subcore_axis_name="s"),
    compiler_params=pltpu.CompilerParams(use_tc_tiling_on_sc=True),
)


_BS = 2048


def _mlp_body(h_ref, m_ref, w0_ref, b0_ref, w1_ref, b1_ref,
              out_ref):
    h = None
    for w in range(_WINDOW):
        g = h_ref[w]
        m = m_ref[:, w:w + 1]
        w0w = w0_ref[:, pl.ds(_EMBED * w, _EMBED)]
        dn = (((1,), (1,)), ((), ()))
        ha = lax.dot_general(g[:, 0:_EMBED], w0w, dn,
                             preferred_element_type=jnp.float32)
        hb = lax.dot_general(g[:, 64:64 + _EMBED], w0w, dn,
                             preferred_element_type=jnp.float32)
        hw = ha + m * (hb - ha)
        h = hw if h is None else h + hw
    h = jnp.tanh(h + b0_ref[...])
    logits = lax.dot_general(h, w1_ref[...], (((1,), (1,)), ((), ())),
                             preferred_element_type=jnp.float32)
    logits = logits + b1_ref[...]
    mx = jnp.max(logits, axis=1, keepdims=True)
    sh = logits - mx
    out_ref[...] = sh - jnp.log(jnp.sum(jnp.exp(sh), axis=1, keepdims=True))


def _tc_mlp(h5, mpk, w0, b0, w1, b1):
    return pl.pallas_call(
        _mlp_body,
        grid=(_BATCH // _BS,),
        in_specs=[
            pl.BlockSpec((_WINDOW, _BS, _LANE), lambda i: (0, i, 0)),
            pl.BlockSpec((_BS, 8), lambda i: (i, 0)),
            pl.BlockSpec((_HIDDEN, _WINDOW * _EMBED), lambda i: (0, 0)),
            pl.BlockSpec((1, _HIDDEN), lambda i: (0, 0)),
            pl.BlockSpec((_TAGS, _HIDDEN), lambda i: (0, 0)),
            pl.BlockSpec((1, _TAGS), lambda i: (0, 0)),
        ],
        out_specs=pl.BlockSpec((_BS, _TAGS), lambda i: (i, 0)),
        out_shape=jax.ShapeDtypeStruct((_BATCH, _TAGS), jnp.float32),
    )(h5, mpk, w0, b0, w1, b1)


def kernel(x, E, W0, b0, W1, b1):
    idx = jnp.asarray(x, jnp.int32).T.reshape(-1)
    q = idx >> _PBBITS
    r = idx & (_PB - 1)
    tpk = ((q >> 1) << _PBBITS) + r
    half = q & 1
    mpk = jnp.pad(
        half.reshape(_WINDOW, _BATCH).T.astype(jnp.float32),
        ((0, 0), (0, 8 - _WINDOW)))

    Epk = _tc_pack(E.T)
    emb = _sc_gather(Epk, tpk)
    h5 = emb.reshape(_WINDOW, _BATCH, _LANE)
    return _tc_mlp(h5, mpk, W0.T,
                   b0.reshape(1, _HIDDEN), W1.T, b1.reshape(1, _TAGS))


# speedup vs baseline: 1.0799x; 1.0797x over previous
import jax
import jax.numpy as jnp
from jax import lax
from jax.experimental import pallas as pl
from jax.experimental.pallas import tpu as pltpu
from jax.experimental.pallas import tpu_sc as plsc

_VOCAB = 1000000
_EMBED = 50
_LANE = 128
_WINDOW = 5
_HIDDEN = 100
_TAGS = 46
_BATCH = 16384

_NIDX = _BATCH * _WINDOW
_NC = 2
_NS = 16
_NW = _NC * _NS
_ROWS_W = _NIDX // _NW
_CHUNK = 320
_NCHUNKS = _ROWS_W // _CHUNK

_PAD_ROWS = 16384


def _pad_body(et_ref, out_ref):
    t = jnp.transpose(et_ref[...], (1, 0))
    out_ref[:, 0:_EMBED] = t
    out_ref[:, _EMBED:_LANE] = jnp.zeros(
        (_PAD_ROWS, _LANE - _EMBED), jnp.float32)


def _tc_pad(et):
    return pl.pallas_call(
        _pad_body,
        grid=(pl.cdiv(_VOCAB, _PAD_ROWS),),
        in_specs=[pl.BlockSpec((_EMBED, _PAD_ROWS), lambda i: (0, i))],
        out_specs=pl.BlockSpec((_PAD_ROWS, _LANE), lambda i: (i, 0)),
        out_shape=jax.ShapeDtypeStruct((_VOCAB, _LANE), jnp.float32),
    )(et)


def _sc_gather_body(table, idxs, out, idx_v, rows0, rows1, sem0, sem1):
    c = lax.axis_index("c")
    s = lax.axis_index("s")
    wid = s * _NC + c
    base = wid * _ROWS_W
    pltpu.sync_copy(idxs.at[pl.ds(base, _ROWS_W)], idx_v)
    bufs = (rows0, rows1)
    sems = (sem0, sem1)
    copies = [None, None]
    copies[0] = pltpu.async_copy(
        table.at[idx_v.at[pl.ds(0, _CHUNK)]], bufs[0], sems[0])
    for j in range(_NCHUNKS):
        cur = j % 2
        nxt = (j + 1) % 2
        if j + 1 < _NCHUNKS:
            copies[nxt] = pltpu.async_copy(
                table.at[idx_v.at[pl.ds((j + 1) * _CHUNK, _CHUNK)]],
                bufs[nxt], sems[nxt])
        copies[cur].wait()
        pltpu.sync_copy(bufs[cur], out.at[pl.ds(base + j * _CHUNK, _CHUNK)])


_sc_gather = pl.kernel(
    _sc_gather_body,
    out_type=jax.ShapeDtypeStruct((_NIDX, _LANE), jnp.float32),
    scratch_types=[
        pltpu.VMEM((_ROWS_W,), jnp.int32),
        pltpu.VMEM((_CHUNK, _LANE), jnp.float32),
        pltpu.VMEM((_CHUNK, _LANE), jnp.float32),
        pltpu.SemaphoreType.DMA,
        pltpu.SemaphoreType.DMA,
    ],
    mesh=plsc.VectorSubcoreMesh(core_axis_name="c", subcore_axis_name="s"),
    compiler_params=pltpu.CompilerParams(use_tc_tiling_on_sc=True),
)


_BS = 2048


def _mlp_body(h_ref, w0_ref, b0_ref, w1_ref, b1_ref, out_ref):
    h = jnp.dot(h_ref[0], w0_ref[0], preferred_element_type=jnp.float32)
    for w in range(1, _WINDOW):
        h = h + jnp.dot(h_ref[w], w0_ref[w],
                        preferred_element_type=jnp.float32)
    h = jnp.tanh(h + b0_ref[...])
    logits = jnp.dot(h, w1_ref[...], preferred_element_type=jnp.float32)
    logits = logits + b1_ref[...]
    m = jnp.max(logits, axis=1, keepdims=True)
    sh = logits - m
    out_ref[...] = sh - jnp.log(jnp.sum(jnp.exp(sh), axis=1, keepdims=True))


def _tc_mlp(h5, w05, b0, w1, b1):
    return pl.pallas_call(
        _mlp_body,
        grid=(_BATCH // _BS,),
        in_specs=[
            pl.BlockSpec((_WINDOW, _BS, _LANE), lambda i: (0, i, 0)),
            pl.BlockSpec((_WINDOW, _LANE, _HIDDEN), lambda i: (0, 0, 0)),
            pl.BlockSpec((1, _HIDDEN), lambda i: (0, 0)),
            pl.BlockSpec((_HIDDEN, _TAGS), lambda i: (0, 0)),
            pl.BlockSpec((1, _TAGS), lambda i: (0, 0)),
        ],
        out_specs=pl.BlockSpec((_BS, _TAGS), lambda i: (i, 0)),
        out_shape=jax.ShapeDtypeStruct((_BATCH, _TAGS), jnp.float32),
    )(h5, w05, b0, w1, b1)


def kernel(x, E, W0, b0, W1, b1):
    idx = jnp.asarray(x, jnp.int32).T.reshape(-1)
    Ep = _tc_pad(E.T)
    W0p = jnp.pad(W0.reshape(_WINDOW, _EMBED, _HIDDEN),
                  ((0, 0), (0, _LANE - _EMBED), (0, 0)))
    emb = _sc_gather(Ep, idx)
    h5 = emb.reshape(_WINDOW, _BATCH, _LANE)
    return _tc_mlp(h5, W0p, b0.reshape(1, _HIDDEN), W1, b1.reshape(1, _TAGS))
